# Initial kernel scaffold; baseline (speedup 1.0000x reference)
import jax
import jax.numpy as jnp
from jax.experimental import pallas as pl


def kernel(context, target, neg_targets, W_in, W_out):
    raise NotImplementedError("write your pallas kernel here")


# speedup vs baseline: 4.5972x; 4.5972x over previous
import functools

import jax
import jax.numpy as jnp
from jax import lax
from jax.experimental import pallas as pl
from jax.experimental.pallas import tpu as pltpu
from jax.experimental.pallas import tpu_sc as plsc

V = 1000000
D = 64
B = 16384
CTX = 20
K = 20

L = 16
NQ = D // L
NW = 32
EPW = B // NW
C = 16
NCHUNK = EPW // C

RC = 4
WC = C * CTX // RC
RT = 4
WT = C * (K + 1) // RT

_mesh = plsc.VectorSubcoreMesh(core_axis_name="c", subcore_axis_name="s")


@functools.partial(
    pl.kernel,
    out_type=jax.ShapeDtypeStruct((B // C, K + 1, C), jnp.float32),
    mesh=_mesh,
    scratch_types=[
        pltpu.VMEM((RC, WC), jnp.int32),
        pltpu.VMEM((RT, WT), jnp.int32),
        pltpu.VMEM((C * CTX, D), jnp.float32),
        pltpu.VMEM((C * (K + 1), D), jnp.float32),
        pltpu.VMEM((K + 1, L), jnp.float32),
        pltpu.SemaphoreType.DMA,
    ],
    compiler_params=pltpu.CompilerParams(
        needs_layout_passes=False, use_tc_tiling_on_sc=False),
)
def _sc_scores(ctx_hbm, tn_hbm, win_hbm, wout_hbm, out_hbm,
               cidx_v, tidx_v, crows_v, trows_v, scores_v, gsem):
    wid = lax.axis_index("s") * 2 + lax.axis_index("c")
    base = wid * EPW
    lane = lax.iota(jnp.int32, L)

    @pl.loop(0, NCHUNK)
    def _chunk(i):
        g = wid * NCHUNK + i
        pltpu.sync_copy(ctx_hbm.at[pl.ds(g * RC, RC), :], cidx_v)
        pltpu.sync_copy(tn_hbm.at[pl.ds(g * RT, RT), :], tidx_v)
        descs = []
        for j in range(RC):
            descs.append(pltpu.async_copy(
                win_hbm.at[cidx_v.at[j]],
                crows_v.at[pl.ds(j * WC, WC), :], gsem))
        for j in range(RT):
            descs.append(pltpu.async_copy(
                wout_hbm.at[tidx_v.at[j]],
                trows_v.at[pl.ds(j * WT, WT), :], gsem))
        for dsc in descs:
            dsc.wait()

        svec = [jnp.zeros((L,), jnp.float32) for _ in range(K + 1)]
        for c in range(C):
            h = [crows_v[c * CTX, pl.ds(q * L, L)] for q in range(NQ)]
            for r in range(1, CTX):
                for q in range(NQ):
                    h[q] = h[q] + crows_v[c * CTX + r, pl.ds(q * L, L)]
            for j in range(K + 1):
                acc = h[0] * trows_v[c * (K + 1) + j, pl.ds(0, L)]
                for q in range(1, NQ):
                    acc = acc + h[q] * trows_v[c * (K + 1) + j, pl.ds(q * L, L)]
                fac = (-1.0 / CTX) if j == 0 else (1.0 / CTX)
                s = plsc.cumsum(acc)[L - 1] * fac
                svec[j] = jnp.where(lane == c, s, svec[j])
        for j in range(K + 1):
            scores_v[j, :] = svec[j]
        pltpu.sync_copy(scores_v, out_hbm.at[g])


def _loss_body(x_ref, o_ref):
    z = x_ref[...]
    sp = jnp.maximum(z, 0.0) + jnp.log1p(jnp.exp(-jnp.abs(z)))
    o_ref[0, 0] = jnp.sum(sp) * (1.0 / B)


_loss_call = pl.pallas_call(
    _loss_body,
    out_shape=jax.ShapeDtypeStruct((1, 1), jnp.float32),
    out_specs=pl.BlockSpec(memory_space=pltpu.SMEM),
)


def kernel(context, target, neg_targets, W_in, W_out):
    ctx2d = context.astype(jnp.int32).reshape(B * CTX // WC, WC)
    tn = jnp.concatenate(
        [target[:, None].astype(jnp.int32),
         neg_targets.astype(jnp.int32)], axis=1)
    tn2d = tn.reshape(B * (K + 1) // WT, WT)
    scores = _sc_scores(ctx2d, tn2d, W_in, W_out)
    loss = _loss_call(scores.reshape((K + 1) * B // 128, 128))
    return loss[0, 0]


# speedup vs baseline: 5.0235x; 1.0927x over previous
import functools

import jax
import jax.numpy as jnp
from jax import lax
from jax.experimental import pallas as pl
from jax.experimental.pallas import tpu as pltpu
from jax.experimental.pallas import tpu_sc as plsc

V = 1000000
D = 64
B = 16384
CTX = 20
K = 20

L = 16
NQ = D // L
NW = 32
EPW = B // NW
C = 8
NCHUNK = EPW // C
GW = 80
NGC = C * CTX // GW

_mesh = plsc.VectorSubcoreMesh(core_axis_name="c", subcore_axis_name="s")


@functools.partial(
    pl.kernel,
    out_type=jax.ShapeDtypeStruct((B // C, K + 1, C), jnp.float32),
    mesh=_mesh,
    scratch_types=[
        pltpu.VMEM((EPW * CTX,), jnp.int32),
        pltpu.VMEM((EPW * K,), jnp.int32),
        pltpu.VMEM((EPW,), jnp.int32),
        pltpu.VMEM((2, C * CTX, D), jnp.float32),
        pltpu.VMEM((2, C * K, D), jnp.float32),
        pltpu.VMEM((2, C, D), jnp.float32),
        pltpu.VMEM((2, K + 1, L), jnp.float32),
        pltpu.SemaphoreType.DMA,
        pltpu.SemaphoreType.DMA,
        pltpu.SemaphoreType.DMA,
    ],
    compiler_params=pltpu.CompilerParams(
        needs_layout_passes=False, use_tc_tiling_on_sc=False),
)
def _sc_scores(ctx_hbm, tgt_hbm, neg_hbm, win_hbm, wout_hbm, out_hbm,
               cidx_v, nidx_v, tidx_v, crows_v, nrows_v, prows_v, scores_v,
               gsem, ssem0, ssem1):
    wid = lax.axis_index("s") * 2 + lax.axis_index("c")
    lane = lax.iota(jnp.int32, L)

    pltpu.sync_copy(ctx_hbm.at[pl.ds(wid * EPW * CTX, EPW * CTX)], cidx_v)
    pltpu.sync_copy(neg_hbm.at[pl.ds(wid * EPW * K, EPW * K)], nidx_v)
    pltpu.sync_copy(tgt_hbm.at[pl.ds(wid * EPW, EPW)], tidx_v)

    def gather_descs(c, b):
        ds = []
        for j in range(NGC):
            ds.append(pltpu.make_async_copy(
                win_hbm.at[cidx_v.at[pl.ds(c * C * CTX + j * GW, GW)]],
                crows_v.at[b, pl.ds(j * GW, GW), :], gsem))
        for j in range(NGC):
            ds.append(pltpu.make_async_copy(
                wout_hbm.at[nidx_v.at[pl.ds(c * C * K + j * GW, GW)]],
                nrows_v.at[b, pl.ds(j * GW, GW), :], gsem))
        ds.append(pltpu.make_async_copy(
            wout_hbm.at[tidx_v.at[pl.ds(c * C, C)]],
            prows_v.at[b], gsem))
        return ds

    def score_desc(c, b):
        g = wid * NCHUNK + c
        sem = ssem0 if b == 0 else ssem1
        return pltpu.make_async_copy(
            scores_v.at[b, :, pl.ds(0, C)], out_hbm.at[g], sem)

    for d in gather_descs(0, 0):
        d.start()

    @pl.loop(0, NCHUNK, step=2)
    def _pair(i):
        for b in range(2):
            chunk = i + b
            for d in gather_descs(chunk, b):
                d.wait()

            @pl.when(chunk + 1 < NCHUNK)
            def _prefetch():
                for d in gather_descs(chunk + 1, 1 - b):
                    d.start()

            @pl.when(chunk >= 2)
            def _drain():
                score_desc(chunk, b).wait()

            svec = [jnp.zeros((L,), jnp.float32) for _ in range(K + 1)]
            for c in range(C):
                h = [crows_v[b, c * CTX, pl.ds(q * L, L)] for q in range(NQ)]
                for r in range(1, CTX):
                    for q in range(NQ):
                        h[q] = h[q] + crows_v[b, c * CTX + r, pl.ds(q * L, L)]
                acc = h[0] * prows_v[b, c, pl.ds(0, L)]
                for q in range(1, NQ):
                    acc = acc + h[q] * prows_v[b, c, pl.ds(q * L, L)]
                s = plsc.cumsum(acc)[L - 1] * (-1.0 / CTX)
                svec[0] = jnp.where(lane == c, s, svec[0])
                for j in range(K):
                    acc = h[0] * nrows_v[b, c * K + j, pl.ds(0, L)]
                    for q in range(1, NQ):
                        acc = acc + h[q] * nrows_v[b, c * K + j, pl.ds(q * L, L)]
                    s = plsc.cumsum(acc)[L - 1] * (1.0 / CTX)
                    svec[j + 1] = jnp.where(lane == c, s, svec[j + 1])
            for j in range(K + 1):
                scores_v[b, j, :] = svec[j]
            score_desc(chunk, b).start()

    for b in range(2):
        score_desc(b, b).wait()


def _loss_body(x_ref, o_ref):
    z = x_ref[...]
    sp = jnp.maximum(z, 0.0) + jnp.log1p(jnp.exp(-jnp.abs(z)))
    o_ref[0, 0] = jnp.sum(sp) * (1.0 / B)


_loss_call = pl.pallas_call(
    _loss_body,
    out_shape=jax.ShapeDtypeStruct((1, 1), jnp.float32),
    out_specs=pl.BlockSpec(memory_space=pltpu.SMEM),
)


def kernel(context, target, neg_targets, W_in, W_out):
    ctx_flat = context.astype(jnp.int32).reshape(-1)
    neg_flat = neg_targets.astype(jnp.int32).reshape(-1)
    tgt = target.astype(jnp.int32)
    scores = _sc_scores(ctx_flat, tgt, neg_flat, W_in, W_out)
    loss = _loss_call(scores.reshape((K + 1) * B // 128, 128))
    return loss[0, 0]
